# baseline (speedup 1.0000x reference)
import functools

import jax
import jax.numpy as jnp
from jax import lax
from jax.experimental import pallas as pl
from jax.experimental.pallas import tpu as pltpu
from jax.experimental.pallas import tpu_sc as plsc

E = 8
TOPK = 2
D = 1024
F = 2048
N = 4096
T = 256
NB = N // T
G = NB + E - 1

NW = 32
CHUNK = N // NW
GROWS = 64
LANES = 16


def _route_body(keys_hbm, hs_hbm, perm_hbm, cnt_hbm,
                keys_v, sidx_v, src_v, rows_v, cnt_v, sem):
    wid = lax.axis_index("s") * 2 + lax.axis_index("c")
    pbase = wid * CHUNK
    lanes = lax.iota(jnp.int32, LANES)

    pltpu.sync_copy(keys_hbm, keys_v)

    def hist_body(v, accs):
        kv = keys_v[pl.ds(v * LANES, LANES)]
        return tuple(accs[e] + jnp.where(kv == e, 1, 0).astype(jnp.int32)
                     for e in range(E))

    accs = lax.fori_loop(0, N // LANES, hist_body,
                         tuple(jnp.zeros((LANES,), jnp.int32) for _ in range(E)))
    totals = [jnp.sum(a) for a in accs]
    offs = [jnp.int32(0)]
    for e in range(E):
        offs.append(offs[-1] + totals[e])

    tv = jnp.zeros((LANES,), jnp.int32)
    for e in range(E):
        tv = jnp.where(lanes == e, totals[e], tv)
    cnt_v[...] = tv

    @pl.when(wid == 0)
    def _():
        pltpu.sync_copy(cnt_v, cnt_hbm)

    for e in range(E):
        lo = offs[e]
        hi = offs[e + 1]

        @pl.when((hi > pbase) & (lo < pbase + CHUNK))
        def _(e=e, lo=lo):
            def pos_body(v, rc):
                kv = keys_v[pl.ds(v * LANES, LANES)]
                m = kv == e
                mi = jnp.where(m, 1, 0).astype(jnp.int32)
                cs = plsc.cumsum(mi)
                p = lo + rc + cs - 1
                inr = m & (p >= pbase) & (p < pbase + CHUNK)
                plsc.store_scatter(sidx_v, [p - pbase], lanes + v * LANES,
                                   mask=inr)
                return rc + jnp.sum(mi)

            lax.fori_loop(0, N // LANES, pos_body, jnp.int32(0))

    for c in range(CHUNK // GROWS):
        for v in range(GROWS // LANES):
            sv = sidx_v[pl.ds(c * GROWS + v * LANES, LANES)]
            src_v[pl.ds(v * LANES, LANES)] = lax.shift_right_logical(sv, 1)
        pltpu.async_copy(hs_hbm.at[src_v], rows_v, sem).wait()
        pltpu.sync_copy(rows_v, perm_hbm.at[pl.ds(pbase + c * GROWS, GROWS)])


@functools.cache
def _make_route():
    return pl.kernel(
        _route_body,
        out_type=(
            jax.ShapeDtypeStruct((N, D), jnp.float32),
            jax.ShapeDtypeStruct((LANES,), jnp.int32),
        ),
        mesh=plsc.VectorSubcoreMesh(core_axis_name="c", subcore_axis_name="s"),
        scratch_types=[
            pltpu.VMEM((N,), jnp.int32),
            pltpu.VMEM((CHUNK,), jnp.int32),
            pltpu.VMEM((GROWS,), jnp.int32),
            pltpu.VMEM((GROWS, D), jnp.float32),
            pltpu.VMEM((LANES,), jnp.int32),
            pltpu.SemaphoreType.DMA,
        ],
        compiler_params=pltpu.CompilerParams(needs_layout_passes=False),
    )


def _gmm_body(gids, tids, offs, x_ref, w1_ref, w2_ref, out_ref):
    s = pl.program_id(0)
    e = gids[s]
    t = tids[s]
    row0 = t * T
    lo = jnp.clip(offs[e] - row0, 0, T)
    hi = jnp.clip(offs[e + 1] - row0, 0, T)

    x = x_ref[...].astype(jnp.bfloat16)
    h = jnp.dot(x, w1_ref[0], preferred_element_type=jnp.float32)
    a = h[:, :F]
    b = h[:, F:]
    inter = ((a * jax.nn.sigmoid(a)) * b).astype(jnp.bfloat16)
    y = jnp.dot(inter, w2_ref[0], preferred_element_type=jnp.float32)

    rows = lax.broadcasted_iota(jnp.int32, (T, 1), 0)
    m = (rows >= lo) & (rows < hi)
    is_first = jnp.logical_or(s == 0, tids[jnp.maximum(s - 1, 0)] != t)

    @pl.when(is_first)
    def _():
        out_ref[...] = jnp.where(m, y, 0.0)

    @pl.when(jnp.logical_not(is_first))
    def _():
        out_ref[...] = jnp.where(m, y, out_ref[...])


def kernel(hidden_states, tokens_per_expert, w1, w2):
    hs = hidden_states.reshape(-1, D)
    keys = tokens_per_expert.reshape(-1)

    permuted, cnt16 = _make_route()(keys, hs)
    counts = cnt16[:E]

    offsets = jnp.concatenate(
        [jnp.zeros((1,), jnp.int32), jnp.cumsum(counts, dtype=jnp.int32)])
    t_start = offsets[:E] // T
    t_end = jnp.where(counts > 0, (offsets[1:] + T - 1) // T, t_start)
    num_t = t_end - t_start
    cum = jnp.cumsum(num_t, dtype=jnp.int32)
    total = cum[E - 1]
    s_eff = jnp.minimum(jnp.arange(G, dtype=jnp.int32), total - 1)
    gids = jnp.searchsorted(cum, s_eff, side="right").astype(jnp.int32)
    tids = (t_start[gids] + (s_eff - (cum[gids] - num_t[gids]))).astype(jnp.int32)

    grid_spec = pltpu.PrefetchScalarGridSpec(
        num_scalar_prefetch=3,
        grid=(G,),
        in_specs=[
            pl.BlockSpec((T, D), lambda s, gids, tids, offs: (tids[s], 0)),
            pl.BlockSpec((1, D, 2 * F), lambda s, gids, tids, offs: (gids[s], 0, 0)),
            pl.BlockSpec((1, F, D), lambda s, gids, tids, offs: (gids[s], 0, 0)),
        ],
        out_specs=pl.BlockSpec((T, D), lambda s, gids, tids, offs: (tids[s], 0)),
    )
    out = pl.pallas_call(
        _gmm_body,
        grid_spec=grid_spec,
        out_shape=jax.ShapeDtypeStruct((N, D), jnp.float32),
    )(gids, tids, offsets, permuted,
      w1.astype(jnp.bfloat16), w2.astype(jnp.bfloat16))
    return out


# speedup vs baseline: 1.2707x; 1.2707x over previous
import functools

import jax
import jax.numpy as jnp
from jax import lax
from jax.experimental import pallas as pl
from jax.experimental.pallas import tpu as pltpu
from jax.experimental.pallas import tpu_sc as plsc

E = 8
TOPK = 2
D = 1024
F = 2048
N = 4096
T = 256
NB = N // T
G = NB + E - 1

NW = 32
CHUNK = N // NW
GROWS = 64
LANES = 16


def _route_body(keys_hbm, hs_hbm, perm_hbm, cnt_hbm,
                keys_v, sidx_v, src_v, rows_v, cnt_v, sem):
    wid = lax.axis_index("s") * 2 + lax.axis_index("c")
    pbase = wid * CHUNK
    lanes = lax.iota(jnp.int32, LANES)

    pltpu.sync_copy(keys_hbm, keys_v)

    def hist_body(v, accs):
        kv = keys_v[pl.ds(v * LANES, LANES)]
        return tuple(accs[e] + jnp.where(kv == e, 1, 0).astype(jnp.int32)
                     for e in range(E))

    accs = lax.fori_loop(0, N // LANES, hist_body,
                         tuple(jnp.zeros((LANES,), jnp.int32) for _ in range(E)))
    totals = [jnp.sum(a) for a in accs]
    offs = [jnp.int32(0)]
    for e in range(E):
        offs.append(offs[-1] + totals[e])

    tv = jnp.zeros((LANES,), jnp.int32)
    for e in range(E):
        tv = jnp.where(lanes == e, totals[e], tv)
    cnt_v[...] = tv

    @pl.when(wid == 0)
    def _():
        pltpu.sync_copy(cnt_v, cnt_hbm)

    for e in range(E):
        lo = offs[e]
        hi = offs[e + 1]

        @pl.when((hi > pbase) & (lo < pbase + CHUNK))
        def _(e=e, lo=lo):
            def pos_body(v, rc):
                kv = keys_v[pl.ds(v * LANES, LANES)]
                m = kv == e
                mi = jnp.where(m, 1, 0).astype(jnp.int32)
                cs = plsc.cumsum(mi)
                p = lo + rc + cs - 1
                inr = m & (p >= pbase) & (p < pbase + CHUNK)
                plsc.store_scatter(sidx_v, [p - pbase], lanes + v * LANES,
                                   mask=inr)
                return rc + jnp.sum(mi)

            lax.fori_loop(0, N // LANES, pos_body, jnp.int32(0))

    for c in range(CHUNK // GROWS):
        for v in range(GROWS // LANES):
            sv = sidx_v[pl.ds(c * GROWS + v * LANES, LANES)]
            src_v[pl.ds(v * LANES, LANES)] = lax.shift_right_logical(sv, 1)
        pltpu.async_copy(hs_hbm.at[src_v], rows_v, sem).wait()
        pltpu.sync_copy(rows_v, perm_hbm.at[pl.ds(pbase + c * GROWS, GROWS)])


@functools.cache
def _make_route():
    return pl.kernel(
        _route_body,
        out_type=(
            jax.ShapeDtypeStruct((N, D), jnp.float32),
            jax.ShapeDtypeStruct((LANES,), jnp.int32),
        ),
        mesh=plsc.VectorSubcoreMesh(core_axis_name="c", subcore_axis_name="s"),
        scratch_types=[
            pltpu.VMEM((N,), jnp.int32),
            pltpu.VMEM((CHUNK,), jnp.int32),
            pltpu.VMEM((GROWS,), jnp.int32),
            pltpu.VMEM((GROWS, D), jnp.float32),
            pltpu.VMEM((LANES,), jnp.int32),
            pltpu.SemaphoreType.DMA,
        ],
        compiler_params=pltpu.CompilerParams(needs_layout_passes=False),
    )


def _gmm_body(gids, tids, offs, x_ref, w1_ref, w2_ref, out_ref):
    s = pl.program_id(0)
    e = gids[s]
    t = tids[s]
    row0 = t * T
    lo = jnp.clip(offs[e] - row0, 0, T)
    hi = jnp.clip(offs[e + 1] - row0, 0, T)

    x = x_ref[...].astype(jnp.bfloat16)
    h = jnp.dot(x, w1_ref[0].astype(jnp.bfloat16),
                preferred_element_type=jnp.float32)
    a = h[:, :F]
    b = h[:, F:]
    inter = ((a * jax.nn.sigmoid(a)) * b).astype(jnp.bfloat16)
    y = jnp.dot(inter, w2_ref[0].astype(jnp.bfloat16),
                preferred_element_type=jnp.float32)

    rows = lax.broadcasted_iota(jnp.int32, (T, 1), 0)
    m = (rows >= lo) & (rows < hi)
    is_first = jnp.logical_or(s == 0, tids[jnp.maximum(s - 1, 0)] != t)

    @pl.when(is_first)
    def _():
        out_ref[...] = jnp.where(m, y, 0.0)

    @pl.when(jnp.logical_not(is_first))
    def _():
        out_ref[...] = jnp.where(m, y, out_ref[...])


def kernel(hidden_states, tokens_per_expert, w1, w2):
    hs = hidden_states.reshape(-1, D)
    keys = tokens_per_expert.reshape(-1)

    permuted, cnt16 = _make_route()(keys, hs)
    counts = cnt16[:E]

    offsets = jnp.concatenate(
        [jnp.zeros((1,), jnp.int32), jnp.cumsum(counts, dtype=jnp.int32)])
    t_start = offsets[:E] // T
    t_end = jnp.where(counts > 0, (offsets[1:] + T - 1) // T, t_start)
    num_t = t_end - t_start
    cum = jnp.cumsum(num_t, dtype=jnp.int32)
    total = cum[E - 1]
    s_eff = jnp.minimum(jnp.arange(G, dtype=jnp.int32), total - 1)
    gids = jnp.searchsorted(cum, s_eff, side="right").astype(jnp.int32)
    tids = (t_start[gids] + (s_eff - (cum[gids] - num_t[gids]))).astype(jnp.int32)

    grid_spec = pltpu.PrefetchScalarGridSpec(
        num_scalar_prefetch=3,
        grid=(G,),
        in_specs=[
            pl.BlockSpec((T, D), lambda s, gids, tids, offs: (tids[s], 0)),
            pl.BlockSpec((1, D, 2 * F), lambda s, gids, tids, offs: (gids[s], 0, 0)),
            pl.BlockSpec((1, F, D), lambda s, gids, tids, offs: (gids[s], 0, 0)),
        ],
        out_specs=pl.BlockSpec((T, D), lambda s, gids, tids, offs: (tids[s], 0)),
    )
    out = pl.pallas_call(
        _gmm_body,
        grid_spec=grid_spec,
        out_shape=jax.ShapeDtypeStruct((N, D), jnp.float32),
    )(gids, tids, offsets, permuted, w1, w2)
    return out


# speedup vs baseline: 1.3072x; 1.0287x over previous
import functools

import jax
import jax.numpy as jnp
from jax import lax
from jax.experimental import pallas as pl
from jax.experimental.pallas import tpu as pltpu
from jax.experimental.pallas import tpu_sc as plsc

E = 8
TOPK = 2
D = 1024
F = 2048
N = 4096
T = 256
TSHIFT = 8
NB = N // T
G = NB + E - 1

NW = 32
CHUNK = N // NW
GROWS = 64
LANES = 16


def _route_body(keys_hbm, hs_hbm, perm_hbm, gids_hbm, tids_hbm, offs_hbm,
                keys_v, sidx_v, src_v, rows_v, meta_v, sem):
    wid = lax.axis_index("s") * 2 + lax.axis_index("c")
    pbase = wid * CHUNK
    lanes = lax.iota(jnp.int32, LANES)

    pltpu.sync_copy(keys_hbm, keys_v)

    def hist_body(v, accs):
        kv = keys_v[pl.ds(v * LANES, LANES)]
        return tuple(accs[e] + jnp.where(kv == e, 1, 0).astype(jnp.int32)
                     for e in range(E))

    accs = lax.fori_loop(0, N // LANES, hist_body,
                         tuple(jnp.zeros((LANES,), jnp.int32) for _ in range(E)))
    totals = [jnp.sum(a) for a in accs]
    offs = [jnp.int32(0)]
    for e in range(E):
        offs.append(offs[-1] + totals[e])

    @pl.when(wid == 0)
    def _():
        t_start = [offs[e] >> TSHIFT for e in range(E)]
        t_end = [jnp.where(totals[e] > 0, (offs[e + 1] + T - 1) >> TSHIFT,
                           t_start[e]) for e in range(E)]
        cum = []
        acc = jnp.int32(0)
        for e in range(E):
            acc = acc + (t_end[e] - t_start[e])
            cum.append(acc)
        total = cum[E - 1]

        ov = jnp.zeros((LANES,), jnp.int32)
        for e in range(E + 1):
            ov = jnp.where(lanes == e, offs[e], ov)
        meta_v[pl.ds(0, LANES)] = ov
        pltpu.sync_copy(meta_v.at[pl.ds(0, LANES)], offs_hbm)

        for chunk in range(2):
            s_eff = jnp.minimum(lanes + chunk * LANES, total - 1)
            gid = jnp.zeros((LANES,), jnp.int32)
            for e in range(E - 1):
                gid = gid + jnp.where(s_eff >= cum[e], 1, 0).astype(jnp.int32)
            ts = jnp.zeros((LANES,), jnp.int32)
            sc = jnp.zeros((LANES,), jnp.int32)
            for e in range(E):
                ts = jnp.where(gid == e, t_start[e], ts)
                sc = jnp.where(gid == e, cum[e] - (t_end[e] - t_start[e]), sc)
            meta_v[pl.ds(chunk * LANES, LANES)] = gid
            meta_v[pl.ds(2 * LANES + chunk * LANES, LANES)] = ts + (s_eff - sc)
        pltpu.sync_copy(meta_v.at[pl.ds(0, 2 * LANES)], gids_hbm)
        pltpu.sync_copy(meta_v.at[pl.ds(2 * LANES, 2 * LANES)], tids_hbm)

    for e in range(E):
        lo = offs[e]
        hi = offs[e + 1]

        @pl.when((hi > pbase) & (lo < pbase + CHUNK))
        def _(e=e, lo=lo):
            def pos_body(v, rc):
                kv = keys_v[pl.ds(v * LANES, LANES)]
                m = kv == e
                mi = jnp.where(m, 1, 0).astype(jnp.int32)
                cs = plsc.cumsum(mi)
                p = lo + rc + cs - 1
                inr = m & (p >= pbase) & (p < pbase + CHUNK)
                plsc.store_scatter(sidx_v, [p - pbase], lanes + v * LANES,
                                   mask=inr)
                return rc + jnp.sum(mi)

            lax.fori_loop(0, N // LANES, pos_body, jnp.int32(0))

    for c in range(CHUNK // GROWS):
        for v in range(GROWS // LANES):
            sv = sidx_v[pl.ds(c * GROWS + v * LANES, LANES)]
            src_v[pl.ds(v * LANES, LANES)] = lax.shift_right_logical(sv, 1)
        pltpu.async_copy(hs_hbm.at[src_v], rows_v, sem).wait()
        pltpu.sync_copy(rows_v, perm_hbm.at[pl.ds(pbase + c * GROWS, GROWS)])


@functools.cache
def _make_route():
    return pl.kernel(
        _route_body,
        out_type=(
            jax.ShapeDtypeStruct((N, D), jnp.float32),
            jax.ShapeDtypeStruct((2 * LANES,), jnp.int32),
            jax.ShapeDtypeStruct((2 * LANES,), jnp.int32),
            jax.ShapeDtypeStruct((LANES,), jnp.int32),
        ),
        mesh=plsc.VectorSubcoreMesh(core_axis_name="c", subcore_axis_name="s"),
        scratch_types=[
            pltpu.VMEM((N,), jnp.int32),
            pltpu.VMEM((CHUNK,), jnp.int32),
            pltpu.VMEM((GROWS,), jnp.int32),
            pltpu.VMEM((GROWS, D), jnp.float32),
            pltpu.VMEM((4 * LANES,), jnp.int32),
            pltpu.SemaphoreType.DMA,
        ],
        compiler_params=pltpu.CompilerParams(needs_layout_passes=False),
    )


def _gmm_body(gids, tids, offs, x_ref, w1_ref, w2_ref, out_ref):
    s = pl.program_id(0)
    e = gids[s]
    t = tids[s]
    row0 = t * T
    lo = jnp.clip(offs[e] - row0, 0, T)
    hi = jnp.clip(offs[e + 1] - row0, 0, T)

    x = x_ref[...]
    h = jnp.dot(x, w1_ref[0], preferred_element_type=jnp.float32)
    a = h[:, :F]
    b = h[:, F:]
    inter = (a * jax.nn.sigmoid(a)) * b
    y = jnp.dot(inter, w2_ref[0], preferred_element_type=jnp.float32)

    rows = lax.broadcasted_iota(jnp.int32, (T, 1), 0)
    m = (rows >= lo) & (rows < hi)
    is_first = jnp.logical_or(s == 0, tids[jnp.maximum(s - 1, 0)] != t)

    @pl.when(is_first)
    def _():
        out_ref[...] = jnp.where(m, y, 0.0)

    @pl.when(jnp.logical_not(is_first))
    def _():
        out_ref[...] = jnp.where(m, y, out_ref[...])


def kernel(hidden_states, tokens_per_expert, w1, w2):
    hs = hidden_states.reshape(-1, D)
    keys = tokens_per_expert.reshape(-1)

    permuted, gids, tids, offsets = _make_route()(keys, hs)

    grid_spec = pltpu.PrefetchScalarGridSpec(
        num_scalar_prefetch=3,
        grid=(G,),
        in_specs=[
            pl.BlockSpec((T, D), lambda s, gids, tids, offs: (tids[s], 0)),
            pl.BlockSpec((1, D, 2 * F), lambda s, gids, tids, offs: (gids[s], 0, 0)),
            pl.BlockSpec((1, F, D), lambda s, gids, tids, offs: (gids[s], 0, 0)),
        ],
        out_specs=pl.BlockSpec((T, D), lambda s, gids, tids, offs: (tids[s], 0)),
    )
    out = pl.pallas_call(
        _gmm_body,
        grid_spec=grid_spec,
        out_shape=jax.ShapeDtypeStruct((N, D), jnp.float32),
    )(gids, tids, offsets, permuted, w1, w2)
    return out


# speedup vs baseline: 1.3178x; 1.0081x over previous
import functools

import jax
import jax.numpy as jnp
from jax import lax
from jax.experimental import pallas as pl
from jax.experimental.pallas import tpu as pltpu
from jax.experimental.pallas import tpu_sc as plsc

E = 8
TOPK = 2
D = 1024
F = 2048
N = 4096
T = 256
TSHIFT = 8
NB = N // T
G = NB + E - 1

NW = 32
CHUNK = N // NW
GROWS = 64
LANES = 16


def _route_body(keys_hbm, hs_hbm, perm_hbm, gids_hbm, tids_hbm, offs_hbm,
                keys_v, sidx_v, src_v, rows_v, meta_v, sem):
    wid = lax.axis_index("s") * 2 + lax.axis_index("c")
    pbase = wid * CHUNK
    lanes = lax.iota(jnp.int32, LANES)

    pltpu.sync_copy(keys_hbm, keys_v)

    def hist_body(v, accs):
        kv = keys_v[pl.ds(v * LANES, LANES)]
        return tuple(accs[e] + jnp.where(kv == e, 1, 0).astype(jnp.int32)
                     for e in range(E))

    accs = lax.fori_loop(0, N // LANES, hist_body,
                         tuple(jnp.zeros((LANES,), jnp.int32) for _ in range(E)))
    totals = [jnp.sum(a) for a in accs]
    offs = [jnp.int32(0)]
    for e in range(E):
        offs.append(offs[-1] + totals[e])

    @pl.when(wid == 0)
    def _():
        t_start = [offs[e] >> TSHIFT for e in range(E)]
        t_end = [jnp.where(totals[e] > 0, (offs[e + 1] + T - 1) >> TSHIFT,
                           t_start[e]) for e in range(E)]
        cum = []
        acc = jnp.int32(0)
        for e in range(E):
            acc = acc + (t_end[e] - t_start[e])
            cum.append(acc)
        total = cum[E - 1]

        ov = jnp.zeros((LANES,), jnp.int32)
        for e in range(E + 1):
            ov = jnp.where(lanes == e, offs[e], ov)
        meta_v[pl.ds(0, LANES)] = ov
        pltpu.sync_copy(meta_v.at[pl.ds(0, LANES)], offs_hbm)

        for chunk in range(2):
            s_eff = jnp.minimum(lanes + chunk * LANES, total - 1)
            gid = jnp.zeros((LANES,), jnp.int32)
            for e in range(E - 1):
                gid = gid + jnp.where(s_eff >= cum[e], 1, 0).astype(jnp.int32)
            ts = jnp.zeros((LANES,), jnp.int32)
            sc = jnp.zeros((LANES,), jnp.int32)
            for e in range(E):
                ts = jnp.where(gid == e, t_start[e], ts)
                sc = jnp.where(gid == e, cum[e] - (t_end[e] - t_start[e]), sc)
            meta_v[pl.ds(chunk * LANES, LANES)] = gid
            meta_v[pl.ds(2 * LANES + chunk * LANES, LANES)] = ts + (s_eff - sc)
        pltpu.sync_copy(meta_v.at[pl.ds(0, 2 * LANES)], gids_hbm)
        pltpu.sync_copy(meta_v.at[pl.ds(2 * LANES, 2 * LANES)], tids_hbm)

    for e in range(E):
        lo = offs[e]
        hi = offs[e + 1]

        @pl.when((hi > pbase) & (lo < pbase + CHUNK))
        def _(e=e, lo=lo):
            def pos_body(v, rc):
                kv = keys_v[pl.ds(v * LANES, LANES)]
                m = kv == e
                mi = jnp.where(m, 1, 0).astype(jnp.int32)
                cs = plsc.cumsum(mi)
                p = lo + rc + cs - 1
                inr = m & (p >= pbase) & (p < pbase + CHUNK)
                plsc.store_scatter(sidx_v, [p - pbase], lanes + v * LANES,
                                   mask=inr)
                return rc + jnp.sum(mi)

            lax.fori_loop(0, N // LANES, pos_body, jnp.int32(0))

    for c in range(CHUNK // GROWS):
        for v in range(GROWS // LANES):
            sv = sidx_v[pl.ds(c * GROWS + v * LANES, LANES)]
            src_v[pl.ds(v * LANES, LANES)] = lax.shift_right_logical(sv, 1)
        pltpu.async_copy(hs_hbm.at[src_v], rows_v, sem).wait()
        pltpu.sync_copy(rows_v, perm_hbm.at[pl.ds(pbase + c * GROWS, GROWS)])


@functools.cache
def _make_route():
    return pl.kernel(
        _route_body,
        out_type=(
            jax.ShapeDtypeStruct((N, D), jnp.float32),
            jax.ShapeDtypeStruct((2 * LANES,), jnp.int32),
            jax.ShapeDtypeStruct((2 * LANES,), jnp.int32),
            jax.ShapeDtypeStruct((LANES,), jnp.int32),
        ),
        mesh=plsc.VectorSubcoreMesh(core_axis_name="c", subcore_axis_name="s"),
        scratch_types=[
            pltpu.VMEM((N,), jnp.int32),
            pltpu.VMEM((CHUNK,), jnp.int32),
            pltpu.VMEM((GROWS,), jnp.int32),
            pltpu.VMEM((GROWS, D), jnp.float32),
            pltpu.VMEM((4 * LANES,), jnp.int32),
            pltpu.SemaphoreType.DMA,
        ],
        compiler_params=pltpu.CompilerParams(needs_layout_passes=False),
    )


def _gmm_body(gids, tids, offs, x_ref, w1_ref, w2_ref, out_ref):
    s = pl.program_id(0)
    e = gids[s]
    t = tids[s]
    row0 = t * T
    lo = jnp.clip(offs[e] - row0, 0, T)
    hi = jnp.clip(offs[e + 1] - row0, 0, T)

    x = x_ref[...]
    h = jnp.dot(x, w1_ref[0], preferred_element_type=jnp.float32)
    a = h[:, :F]
    b = h[:, F:]
    inter = (a * jax.nn.sigmoid(a)) * b
    y = jnp.dot(inter, w2_ref[0], preferred_element_type=jnp.float32)

    rows = lax.broadcasted_iota(jnp.int32, (T, 1), 0)
    m = (rows >= lo) & (rows < hi)
    is_first = jnp.logical_or(s == 0, tids[jnp.maximum(s - 1, 0)] != t)

    @pl.when(is_first)
    def _():
        out_ref[...] = jnp.where(m, y, 0.0)

    @pl.when(jnp.logical_not(is_first))
    def _():
        out_ref[...] = jnp.where(m, y, out_ref[...])


def kernel(hidden_states, tokens_per_expert, w1, w2):
    hs = hidden_states.reshape(-1, D)
    keys = tokens_per_expert.reshape(-1)

    permuted, gids, tids, offsets = _make_route()(keys, hs)

    grid_spec = pltpu.PrefetchScalarGridSpec(
        num_scalar_prefetch=3,
        grid=(G,),
        in_specs=[
            pl.BlockSpec((T, D), lambda s, gids, tids, offs: (tids[s], 0)),
            pl.BlockSpec((1, D, 2 * F), lambda s, gids, tids, offs: (gids[s], 0, 0)),
            pl.BlockSpec((1, F, D), lambda s, gids, tids, offs: (gids[s], 0, 0)),
        ],
        out_specs=pl.BlockSpec((T, D), lambda s, gids, tids, offs: (tids[s], 0)),
    )
    out = pl.pallas_call(
        _gmm_body,
        grid_spec=grid_spec,
        out_shape=jax.ShapeDtypeStruct((N, D), jnp.float32),
        compiler_params=pltpu.CompilerParams(
            vmem_limit_bytes=100 * 1024 * 1024),
    )(gids, tids, offsets, permuted, w1, w2)
    return out


# speedup vs baseline: 1.3367x; 1.0143x over previous
import functools

import jax
import jax.numpy as jnp
from jax import lax
from jax.experimental import pallas as pl
from jax.experimental.pallas import tpu as pltpu
from jax.experimental.pallas import tpu_sc as plsc

E = 8
TOPK = 2
D = 1024
F = 2048
N = 4096
T = 256
TSHIFT = 8
NB = N // T
G = NB + E - 1

NW = 32
CHUNK = N // NW
GROWS = 32
LANES = 16


def _route_body(keys_hbm, hs_hbm, perm_hbm, gids_hbm, tids_hbm, offs_hbm,
                keys_v, sidx_v, src_v, rows_a, rows_b, meta_v, sem_a, sem_b):
    wid = lax.axis_index("s") * 2 + lax.axis_index("c")
    pbase = wid * CHUNK
    lanes = lax.iota(jnp.int32, LANES)

    pltpu.sync_copy(keys_hbm, keys_v)

    def hist_body(v, accs):
        kv = keys_v[pl.ds(v * LANES, LANES)]
        return tuple(accs[e] + jnp.where(kv == e, 1, 0).astype(jnp.int32)
                     for e in range(E))

    accs = lax.fori_loop(0, N // LANES, hist_body,
                         tuple(jnp.zeros((LANES,), jnp.int32) for _ in range(E)))
    totals = [jnp.sum(a) for a in accs]
    offs = [jnp.int32(0)]
    for e in range(E):
        offs.append(offs[-1] + totals[e])

    @pl.when(wid == 0)
    def _():
        t_start = [offs[e] >> TSHIFT for e in range(E)]
        t_end = [jnp.where(totals[e] > 0, (offs[e + 1] + T - 1) >> TSHIFT,
                           t_start[e]) for e in range(E)]
        cum = []
        acc = jnp.int32(0)
        for e in range(E):
            acc = acc + (t_end[e] - t_start[e])
            cum.append(acc)
        total = cum[E - 1]

        ov = jnp.zeros((LANES,), jnp.int32)
        for e in range(E + 1):
            ov = jnp.where(lanes == e, offs[e], ov)
        meta_v[pl.ds(0, LANES)] = ov
        pltpu.sync_copy(meta_v.at[pl.ds(0, LANES)], offs_hbm)

        for chunk in range(2):
            s_eff = jnp.minimum(lanes + chunk * LANES, total - 1)
            gid = jnp.zeros((LANES,), jnp.int32)
            for e in range(E - 1):
                gid = gid + jnp.where(s_eff >= cum[e], 1, 0).astype(jnp.int32)
            ts = jnp.zeros((LANES,), jnp.int32)
            sc = jnp.zeros((LANES,), jnp.int32)
            for e in range(E):
                ts = jnp.where(gid == e, t_start[e], ts)
                sc = jnp.where(gid == e, cum[e] - (t_end[e] - t_start[e]), sc)
            meta_v[pl.ds(chunk * LANES, LANES)] = gid
            meta_v[pl.ds(2 * LANES + chunk * LANES, LANES)] = ts + (s_eff - sc)
        pltpu.sync_copy(meta_v.at[pl.ds(0, 2 * LANES)], gids_hbm)
        pltpu.sync_copy(meta_v.at[pl.ds(2 * LANES, 2 * LANES)], tids_hbm)

    for e in range(E):
        lo = offs[e]
        hi = offs[e + 1]

        @pl.when((hi > pbase) & (lo < pbase + CHUNK))
        def _(e=e, lo=lo, hi=hi):
            needed = jnp.minimum(hi, pbase + CHUNK) - lo

            def pos_cond(carry):
                v, rc = carry
                return jnp.logical_and(v < N // LANES, rc < needed)

            def pos_body(carry):
                v, rc = carry
                kv = keys_v[pl.ds(v * LANES, LANES)]
                m = kv == e
                mi = jnp.where(m, 1, 0).astype(jnp.int32)
                cs = plsc.cumsum(mi)
                p = lo + rc + cs - 1
                inr = m & (p >= pbase) & (p < pbase + CHUNK)
                plsc.store_scatter(sidx_v, [p - pbase], lanes + v * LANES,
                                   mask=inr)
                return (v + 1, rc + jnp.sum(mi))

            lax.while_loop(pos_cond, pos_body, (jnp.int32(0), jnp.int32(0)))

    for v in range(CHUNK // LANES):
        sv = sidx_v[pl.ds(v * LANES, LANES)]
        src_v[pl.ds(v * LANES, LANES)] = lax.shift_right_logical(sv, 1)

    bufs = (rows_a, rows_b)
    sems = (sem_a, sem_b)
    nch = CHUNK // GROWS
    copies = [None] * nch
    for c in range(nch):
        copies[c] = pltpu.async_copy(
            hs_hbm.at[src_v.at[pl.ds(c * GROWS, GROWS)]], bufs[c % 2],
            sems[c % 2])
        if c > 0:
            copies[c - 1].wait()
            pltpu.sync_copy(bufs[(c - 1) % 2],
                            perm_hbm.at[pl.ds(pbase + (c - 1) * GROWS, GROWS)])
    copies[nch - 1].wait()
    pltpu.sync_copy(bufs[(nch - 1) % 2],
                    perm_hbm.at[pl.ds(pbase + (nch - 1) * GROWS, GROWS)])


@functools.cache
def _make_route():
    return pl.kernel(
        _route_body,
        out_type=(
            jax.ShapeDtypeStruct((N, D), jnp.float32),
            jax.ShapeDtypeStruct((2 * LANES,), jnp.int32),
            jax.ShapeDtypeStruct((2 * LANES,), jnp.int32),
            jax.ShapeDtypeStruct((LANES,), jnp.int32),
        ),
        mesh=plsc.VectorSubcoreMesh(core_axis_name="c", subcore_axis_name="s"),
        scratch_types=[
            pltpu.VMEM((N,), jnp.int32),
            pltpu.VMEM((CHUNK,), jnp.int32),
            pltpu.VMEM((CHUNK,), jnp.int32),
            pltpu.VMEM((GROWS, D), jnp.float32),
            pltpu.VMEM((GROWS, D), jnp.float32),
            pltpu.VMEM((4 * LANES,), jnp.int32),
            pltpu.SemaphoreType.DMA,
            pltpu.SemaphoreType.DMA,
        ],
        compiler_params=pltpu.CompilerParams(needs_layout_passes=False),
    )


def _gmm_body(gids, tids, offs, x_ref, w1_ref, w2_ref, out_ref):
    s = pl.program_id(0)
    e = gids[s]
    t = tids[s]
    row0 = t * T
    lo = jnp.clip(offs[e] - row0, 0, T)
    hi = jnp.clip(offs[e + 1] - row0, 0, T)

    x = x_ref[...]
    h = jnp.dot(x, w1_ref[0], preferred_element_type=jnp.float32)
    a = h[:, :F]
    b = h[:, F:]
    inter = (a * jax.nn.sigmoid(a)) * b
    y = jnp.dot(inter, w2_ref[0], preferred_element_type=jnp.float32)

    rows = lax.broadcasted_iota(jnp.int32, (T, 1), 0)
    m = (rows >= lo) & (rows < hi)
    is_first = jnp.logical_or(s == 0, tids[jnp.maximum(s - 1, 0)] != t)

    @pl.when(is_first)
    def _():
        out_ref[...] = jnp.where(m, y, 0.0)

    @pl.when(jnp.logical_not(is_first))
    def _():
        out_ref[...] = jnp.where(m, y, out_ref[...])


def kernel(hidden_states, tokens_per_expert, w1, w2):
    hs = hidden_states.reshape(-1, D)
    keys = tokens_per_expert.reshape(-1)

    permuted, gids, tids, offsets = _make_route()(keys, hs)

    grid_spec = pltpu.PrefetchScalarGridSpec(
        num_scalar_prefetch=3,
        grid=(G,),
        in_specs=[
            pl.BlockSpec((T, D), lambda s, gids, tids, offs: (tids[s], 0)),
            pl.BlockSpec((1, D, 2 * F), lambda s, gids, tids, offs: (gids[s], 0, 0)),
            pl.BlockSpec((1, F, D), lambda s, gids, tids, offs: (gids[s], 0, 0)),
        ],
        out_specs=pl.BlockSpec((T, D), lambda s, gids, tids, offs: (tids[s], 0)),
    )
    out = pl.pallas_call(
        _gmm_body,
        grid_spec=grid_spec,
        out_shape=jax.ShapeDtypeStruct((N, D), jnp.float32),
        compiler_params=pltpu.CompilerParams(
            vmem_limit_bytes=100 * 1024 * 1024),
    )(gids, tids, offsets, permuted, w1, w2)
    return out
